# baseline (speedup 1.0000x reference)
import functools

import jax
import jax.numpy as jnp
from jax import lax
from jax.experimental import pallas as pl
from jax.experimental.pallas import tpu as pltpu
from jax.experimental.pallas import tpu_sc as plsc

_N = 10000
_D = 256
_E = 160000
_NPAD = 10240
_HALF = 128
_NC = 2
_NS = 16
_K = 128
_NOP = _E // _K
_OPT_E = 80
_OPT_D = 40
_OPROWS = 1280
_ROWS_PER_SUB = _NPAD // _NS
_NSLOT = 2
_OPC = 40
_NCH = _OPT_E // _OPC
_TAIL = _N % _K


def _deg_body(dst2_hbm, out_hbm, didx2, ones, zrow, deg_sh, sem):
    c = lax.axis_index("c")
    s = lax.axis_index("s")
    for u in range(8):
        ones[0, pl.ds(u * 16, 16)] = jnp.full((16,), 1.0, jnp.float32)
        zrow[0, pl.ds(u * 16, 16)] = jnp.zeros((16,), jnp.float32)
    def zbody(m, _):
        pltpu.sync_copy(zrow.at[0], deg_sh.at[pl.ds(s * _ROWS_PER_SUB + m * _HALF, _HALF)])
        return 0
    lax.fori_loop(0, _ROWS_PER_SUB // _HALF, zbody, 0)

    w = s * _NC + c
    base = w * _OPT_D
    nops = jnp.minimum(_OPT_D, _NOP - base)
    pltpu.sync_copy(dst2_hbm.at[pl.ds(base, _OPT_D), :], didx2)
    plsc.subcore_barrier()

    def fire(j, _):
        @pl.when(j < nops)
        def _():
            pltpu.async_copy(ones.at[0], deg_sh.at[didx2.at[j]], sem, add=True)
        return 0
    lax.fori_loop(0, _OPT_D, fire, 0)
    def drain(j, _):
        @pl.when(j < nops)
        def _():
            pltpu.make_async_copy(ones.at[0], deg_sh.at[didx2.at[0]], sem).wait()
        return 0
    lax.fori_loop(0, _OPT_D, drain, 0)
    plsc.subcore_barrier()

    def wbody(m, _):
        off = s * _ROWS_PER_SUB + m * _HALF
        pltpu.sync_copy(deg_sh.at[pl.ds(off, _HALF)], zrow.at[0])
        pltpu.sync_copy(zrow.at[0], out_hbm.at[pl.ds(c * _NPAD + off, _HALF)])
        return 0
    lax.fori_loop(0, _ROWS_PER_SUB // _HALF, wbody, 0)


@functools.partial(
    pl.kernel,
    out_type=jax.ShapeDtypeStruct((_NC * _NPAD,), jnp.float32),
    mesh=plsc.VectorSubcoreMesh(core_axis_name="c", subcore_axis_name="s"),
    scratch_types=[
        pltpu.VMEM((_OPT_D, _K), jnp.int32),
        pltpu.VMEM((1, _HALF), jnp.float32),
        pltpu.VMEM((1, _HALF), jnp.float32),
        pltpu.VMEM_SHARED((_NPAD,), jnp.float32),
        pltpu.SemaphoreType.DMA,
    ],
)
def _deg_call(dst2_hbm, out_hbm, didx2, ones, zrow, deg_sh, sem):
    _deg_body(dst2_hbm, out_hbm, didx2, ones, zrow, deg_sh, sem)


def _ab_tc(x_ref, w_ref, p0_ref, p1_ref, hp2_ref, dinv_ref):
    h = jnp.dot(x_ref[...], w_ref[...], preferred_element_type=jnp.float32)
    deg = p0_ref[...] + p1_ref[...] + 1.0
    dinv = lax.rsqrt(deg)
    hp = h * dinv
    hp2_ref[0] = hp[:, :_HALF]
    hp2_ref[1] = hp[:, _HALF:]
    dinv_ref[...] = dinv


def _edge_body(src2_hbm, dst2_hbm, hcat_hbm, dv_hbm, b2_hbm, out_hbm,
               sidx2, didx2, rows, dv, bbuf, acc_sh, gsems, ssems):
    c = lax.axis_index("c")
    s = lax.axis_index("s")
    base = s * _OPT_E

    def zfill(i, _):
        for u in range(8):
            rows[0][i, pl.ds(u * 16, 16)] = jnp.zeros((16,), jnp.float32)
        return 0
    lax.fori_loop(0, _K, zfill, 0)
    def zbody(m, _):
        pltpu.sync_copy(rows[0], acc_sh.at[pl.ds(s * _ROWS_PER_SUB + m * _K, _K)])
        return 0
    lax.fori_loop(0, _ROWS_PER_SUB // _K, zbody, 0)

    cbase = c * _N

    def g_start(j, p):
        pltpu.async_copy(hcat_hbm.at[sidx2.at[j]], rows[p], gsems[p])
    def g_wait(p):
        pltpu.make_async_copy(hcat_hbm.at[sidx2.at[0]], rows[p], gsems[p]).wait()
    def s_start(j, p):
        pltpu.async_copy(rows[p], acc_sh.at[didx2.at[j]], ssems[p], add=True)
    def s_wait(p):
        pltpu.make_async_copy(rows[p], acc_sh.at[didx2.at[0]], ssems[p]).wait()

    plsc.subcore_barrier()

    for ch in range(_NCH):
        cb = base + ch * _OPC
        nops = jnp.minimum(_OPC, jnp.maximum(0, _NOP - cb))
        pltpu.sync_copy(src2_hbm.at[pl.ds(cb, _OPC), :], sidx2)
        pltpu.sync_copy(dst2_hbm.at[pl.ds(cb, _OPC), :], didx2)
        def tbody(j, _):
            for u in range(8):
                sidx2[j, pl.ds(u * 16, 16)] = sidx2[j, pl.ds(u * 16, 16)] + cbase
            return 0
        lax.fori_loop(0, _OPC, tbody, 0)

        for q in range(_NSLOT):
            @pl.when(q < nops)
            def _(q=q):
                g_start(q, q)

        def lbody(t, _):
            j0 = t * _NSLOT
            for q in range(_NSLOT):
                @pl.when(j0 + q < nops)
                def _(q=q):
                    g_wait(q)
                    s_start(j0 + q, q)
            for q in range(_NSLOT):
                @pl.when(j0 + q < nops)
                def _(q=q):
                    s_wait(q)
                @pl.when(j0 + q + _NSLOT < nops)
                def _(q=q):
                    g_start(j0 + q + _NSLOT, q)
            return 0
        lax.fori_loop(0, (_OPC + _NSLOT - 1) // _NSLOT, lbody, 0)
    plsc.subcore_barrier()

    pltpu.sync_copy(b2_hbm.at[pl.ds(c, 1), :], bbuf)
    bu = [bbuf[0, pl.ds(u * 16, 16)] for u in range(8)]

    def do_chunk(off, offd, sz):
        off = pl.multiple_of(off, _K)
        offd = pl.multiple_of(offd, _K // 8)
        pltpu.sync_copy(acc_sh.at[pl.ds(off, sz)], rows[0].at[pl.ds(0, sz)])
        pltpu.sync_copy(hcat_hbm.at[pl.ds(pl.multiple_of(cbase + off, _K), sz)],
                        rows[1].at[pl.ds(0, sz)])
        pltpu.sync_copy(dv_hbm.at[pl.ds(offd, _K // 8), :], dv)
        def fbody(i, _):
            v = dv[i // 8, pl.ds((i % 8) * 16, 16)]
            for u in range(8):
                a = rows[0][i, pl.ds(u * 16, 16)]
                hpv = rows[1][i, pl.ds(u * 16, 16)]
                rows[0][i, pl.ds(u * 16, 16)] = (a + hpv) * v + bu[u]
            return 0
        lax.fori_loop(0, sz, fbody, 0)
        pltpu.sync_copy(rows[0].at[pl.ds(0, sz)],
                        out_hbm.at[pl.ds(off, sz), pl.ds(c * _HALF, _HALF)])

    for m in range(_ROWS_PER_SUB // _K):
        off = s * _ROWS_PER_SUB + m * _K
        offd = s * (_ROWS_PER_SUB // 8) + m * (_K // 8)
        @pl.when(off + _K <= _N)
        def _(off=off, offd=offd):
            do_chunk(off, offd, _K)
        @pl.when((off < _N) & (off + _K > _N))
        def _(off=off, offd=offd):
            do_chunk(off, offd, _TAIL)


@functools.partial(
    pl.kernel,
    out_type=jax.ShapeDtypeStruct((_N, _D), jnp.float32),
    mesh=plsc.VectorSubcoreMesh(core_axis_name="c", subcore_axis_name="s"),
    scratch_types=[
        pltpu.VMEM((_OPC, _K), jnp.int32),
        pltpu.VMEM((_OPC, _K), jnp.int32),
        [pltpu.VMEM((_K, _HALF), jnp.float32)] * _NSLOT,
        pltpu.VMEM((_K // 8, _HALF), jnp.float32),
        pltpu.VMEM((1, _HALF), jnp.float32),
        pltpu.VMEM_SHARED((_NPAD, _HALF), jnp.float32),
        [pltpu.SemaphoreType.DMA] * _NSLOT,
        [pltpu.SemaphoreType.DMA] * _NSLOT,
    ],
)
def _edge_call(src2_hbm, dst2_hbm, hcat_hbm, dv_hbm, b2_hbm, out_hbm,
               sidx2, didx2, rows, dv, bbuf, acc_sh, gsems, ssems):
    _edge_body(src2_hbm, dst2_hbm, hcat_hbm, dv_hbm, b2_hbm, out_hbm,
               sidx2, didx2, rows, dv, bbuf, acc_sh, gsems, ssems)


_BN = 2000


def kernel(x, edge_index, W, b):
    pad = _OPROWS * _K - _E
    src2 = jnp.pad(edge_index[0], (0, pad)).reshape(_OPROWS, _K)
    dst2 = jnp.pad(edge_index[1], (0, pad)).reshape(_OPROWS, _K)

    degflat = _deg_call(dst2)
    p0 = degflat[:_N].reshape(_N, 1)
    p1 = degflat[_NPAD:_NPAD + _N].reshape(_N, 1)

    hp2, dinvrep = pl.pallas_call(
        _ab_tc,
        grid=(_N // _BN,),
        in_specs=[
            pl.BlockSpec((_BN, _D), lambda i: (i, 0)),
            pl.BlockSpec((_D, _D), lambda i: (0, 0)),
            pl.BlockSpec((_BN, 1), lambda i: (i, 0)),
            pl.BlockSpec((_BN, 1), lambda i: (i, 0)),
        ],
        out_specs=[
            pl.BlockSpec((2, _BN, _HALF), lambda i: (0, i, 0)),
            pl.BlockSpec((_BN, 1), lambda i: (i, 0)),
        ],
        out_shape=[
            jax.ShapeDtypeStruct((2, _N, _HALF), jnp.float32),
            jax.ShapeDtypeStruct((_N, 1), jnp.float32),
        ],
    )(x, W, p0, p1)

    hcat = hp2.reshape(2 * _N, _HALF)
    b2 = b.reshape(_NC, _HALF)
    dpad = jnp.pad(dinvrep, ((0, _NPAD - _N), (0, 0)))
    dvr = jnp.broadcast_to(
        dpad.reshape(_NPAD // 8, 8, 1), (_NPAD // 8, 8, 16)).reshape(_NPAD // 8, _HALF)
    return _edge_call(src2, dst2, hcat, dvr, b2)


# speedup vs baseline: 1.1574x; 1.1574x over previous
import functools

import jax
import jax.numpy as jnp
from jax import lax
from jax.experimental import pallas as pl
from jax.experimental.pallas import tpu as pltpu
from jax.experimental.pallas import tpu_sc as plsc

_N = 10000
_D = 256
_E = 160000
_NPAD = 10240
_HALF = 128
_NC = 2
_NS = 16
_K = 128
_NOP = _E // _K
_OPT_E = 80
_OPT_D = 40
_OPROWS = 1280
_ROWS_PER_SUB = _NPAD // _NS
_NSLOT = 2
_OPC = 40
_NCH = _OPT_E // _OPC
_TAIL = _N % _K


def _deg_body(dst2_hbm, out_hbm, didx2, ones, zrow, deg_sh, sem):
    c = lax.axis_index("c")
    s = lax.axis_index("s")
    for u in range(8):
        ones[0, pl.ds(u * 16, 16)] = jnp.full((16,), 1.0, jnp.float32)
        zrow[0, pl.ds(u * 16, 16)] = jnp.zeros((16,), jnp.float32)
    def zbody(m, _):
        pltpu.sync_copy(zrow.at[0], deg_sh.at[pl.ds(s * _ROWS_PER_SUB + m * _HALF, _HALF)])
        return 0
    lax.fori_loop(0, _ROWS_PER_SUB // _HALF, zbody, 0)

    w = s * _NC + c
    base = w * _OPT_D
    nops = jnp.minimum(_OPT_D, _NOP - base)
    pltpu.sync_copy(dst2_hbm.at[pl.ds(base, _OPT_D), :], didx2)
    plsc.subcore_barrier()

    def fire(j, _):
        @pl.when(j < nops)
        def _():
            pltpu.async_copy(ones.at[0], deg_sh.at[didx2.at[j]], sem, add=True)
        return 0
    lax.fori_loop(0, _OPT_D, fire, 0)
    def drain(j, _):
        @pl.when(j < nops)
        def _():
            pltpu.make_async_copy(ones.at[0], deg_sh.at[didx2.at[0]], sem).wait()
        return 0
    lax.fori_loop(0, _OPT_D, drain, 0)
    plsc.subcore_barrier()

    def wbody(m, _):
        off = s * _ROWS_PER_SUB + m * _HALF
        pltpu.sync_copy(deg_sh.at[pl.ds(off, _HALF)], zrow.at[0])
        pltpu.sync_copy(zrow.at[0], out_hbm.at[pl.ds(c * _NPAD + off, _HALF)])
        return 0
    lax.fori_loop(0, _ROWS_PER_SUB // _HALF, wbody, 0)


@functools.partial(
    pl.kernel,
    out_type=jax.ShapeDtypeStruct((_NC * _NPAD,), jnp.float32),
    mesh=plsc.VectorSubcoreMesh(core_axis_name="c", subcore_axis_name="s"),
    scratch_types=[
        pltpu.VMEM((_OPT_D, _K), jnp.int32),
        pltpu.VMEM((1, _HALF), jnp.float32),
        pltpu.VMEM((1, _HALF), jnp.float32),
        pltpu.VMEM_SHARED((_NPAD,), jnp.float32),
        pltpu.SemaphoreType.DMA,
    ],
)
def _deg_call(dst2_hbm, out_hbm, didx2, ones, zrow, deg_sh, sem):
    _deg_body(dst2_hbm, out_hbm, didx2, ones, zrow, deg_sh, sem)


def _ab_tc(x_ref, w_ref, p0_ref, p1_ref, hp2_ref, dinv_ref):
    h = jnp.dot(x_ref[...], w_ref[...], preferred_element_type=jnp.float32)
    deg = p0_ref[...] + p1_ref[...] + 1.0
    dinv = lax.rsqrt(deg)
    hp = h * dinv
    hp2_ref[0] = hp[:, :_HALF]
    hp2_ref[1] = hp[:, _HALF:]
    dinv_ref[...] = dinv


def _edge_body(src2_hbm, dst2_hbm, hcat_hbm, out_hbm, sidx2, didx2, rows, acc_sh, gsems, ssems):
    c = lax.axis_index("c")
    s = lax.axis_index("s")
    base = s * _OPT_E
    cbase = c * _N

    def ibody(m, _):
        off = s * _ROWS_PER_SUB + m * _K
        @pl.when(off + _K <= _N)
        def _():
            pltpu.sync_copy(
                hcat_hbm.at[pl.ds(pl.multiple_of(cbase + off, _K), _K)], rows[0])
            pltpu.sync_copy(rows[0], acc_sh.at[pl.ds(off, _K), :])
        @pl.when((off < _N) & (off + _K > _N))
        def _():
            pltpu.sync_copy(
                hcat_hbm.at[pl.ds(pl.multiple_of(cbase + off, _K), _TAIL)],
                rows[0].at[pl.ds(0, _TAIL), :])
            pltpu.sync_copy(rows[0].at[pl.ds(0, _TAIL), :],
                            acc_sh.at[pl.ds(off, _TAIL), :])
        return 0
    lax.fori_loop(0, _ROWS_PER_SUB // _K, ibody, 0)

    def g_start(j, p):
        pltpu.async_copy(hcat_hbm.at[sidx2.at[j]], rows[p], gsems[p])
    def g_wait(p):
        pltpu.make_async_copy(hcat_hbm.at[sidx2.at[0]], rows[p], gsems[p]).wait()
    def s_start(j, p):
        pltpu.async_copy(rows[p], acc_sh.at[didx2.at[j]], ssems[p], add=True)
    def s_wait(p):
        pltpu.make_async_copy(rows[p], acc_sh.at[didx2.at[0]], ssems[p]).wait()

    plsc.subcore_barrier()

    for ch in range(_NCH):
        cb = base + ch * _OPC
        nops = jnp.minimum(_OPC, jnp.maximum(0, _NOP - cb))
        pltpu.sync_copy(src2_hbm.at[pl.ds(cb, _OPC), :], sidx2)
        pltpu.sync_copy(dst2_hbm.at[pl.ds(cb, _OPC), :], didx2)
        def tbody(j, _):
            for u in range(8):
                sidx2[j, pl.ds(u * 16, 16)] = sidx2[j, pl.ds(u * 16, 16)] + cbase
            return 0
        lax.fori_loop(0, _OPC, tbody, 0)

        for q in range(_NSLOT):
            @pl.when(q < nops)
            def _(q=q):
                g_start(q, q)

        def lbody(t, _):
            j0 = t * _NSLOT
            for q in range(_NSLOT):
                @pl.when(j0 + q < nops)
                def _(q=q):
                    g_wait(q)
                    s_start(j0 + q, q)
            for q in range(_NSLOT):
                @pl.when(j0 + q < nops)
                def _(q=q):
                    s_wait(q)
                @pl.when(j0 + q + _NSLOT < nops)
                def _(q=q):
                    g_start(j0 + q + _NSLOT, q)
            return 0
        lax.fori_loop(0, (_OPC + _NSLOT - 1) // _NSLOT, lbody, 0)
    plsc.subcore_barrier()

    def wbody(m, _):
        off = s * _ROWS_PER_SUB + m * _K
        pltpu.sync_copy(acc_sh.at[pl.ds(off, _K), :], rows[0])
        pltpu.sync_copy(rows[0], out_hbm.at[pl.ds(c * _NPAD + off, _K), :])
        return 0
    lax.fori_loop(0, _ROWS_PER_SUB // _K, wbody, 0)


@functools.partial(
    pl.kernel,
    out_type=jax.ShapeDtypeStruct((_NC * _NPAD, _HALF), jnp.float32),
    mesh=plsc.VectorSubcoreMesh(core_axis_name="c", subcore_axis_name="s"),
    scratch_types=[
        pltpu.VMEM((_OPC, _K), jnp.int32),
        pltpu.VMEM((_OPC, _K), jnp.int32),
        [pltpu.VMEM((_K, _HALF), jnp.float32)] * _NSLOT,
        pltpu.VMEM_SHARED((_NPAD, _HALF), jnp.float32),
        [pltpu.SemaphoreType.DMA] * _NSLOT,
        [pltpu.SemaphoreType.DMA] * _NSLOT,
    ],
)
def _edge_call(src2_hbm, dst2_hbm, hcat_hbm, out_hbm, sidx2, didx2, rows, acc_sh, gsems, ssems):
    _edge_body(src2_hbm, dst2_hbm, hcat_hbm, out_hbm, sidx2, didx2, rows, acc_sh, gsems, ssems)


def _fin_tc(acc_ref, dinv_ref, b_ref, o_ref):
    dinv = dinv_ref[...]
    o0 = acc_ref[0] * dinv
    o1 = acc_ref[1] * dinv
    o_ref[...] = jnp.concatenate([o0, o1], axis=1) + b_ref[...]


_BN = 2000


def kernel(x, edge_index, W, b):
    pad = _OPROWS * _K - _E
    src2 = jnp.pad(edge_index[0], (0, pad)).reshape(_OPROWS, _K)
    dst2 = jnp.pad(edge_index[1], (0, pad)).reshape(_OPROWS, _K)

    degflat = _deg_call(dst2)
    p0 = degflat[:_N].reshape(_N, 1)
    p1 = degflat[_NPAD:_NPAD + _N].reshape(_N, 1)

    hp2, dinv = pl.pallas_call(
        _ab_tc,
        grid=(_N // _BN,),
        in_specs=[
            pl.BlockSpec((_BN, _D), lambda i: (i, 0)),
            pl.BlockSpec((_D, _D), lambda i: (0, 0)),
            pl.BlockSpec((_BN, 1), lambda i: (i, 0)),
            pl.BlockSpec((_BN, 1), lambda i: (i, 0)),
        ],
        out_specs=[
            pl.BlockSpec((2, _BN, _HALF), lambda i: (0, i, 0)),
            pl.BlockSpec((_BN, 1), lambda i: (i, 0)),
        ],
        out_shape=[
            jax.ShapeDtypeStruct((2, _N, _HALF), jnp.float32),
            jax.ShapeDtypeStruct((_N, 1), jnp.float32),
        ],
    )(x, W, p0, p1)

    hcat = hp2.reshape(2 * _N, _HALF)
    accflat = _edge_call(src2, dst2, hcat)
    acc = accflat.reshape(2, _NPAD, _HALF)

    out = pl.pallas_call(
        _fin_tc,
        grid=(_N // _BN,),
        in_specs=[
            pl.BlockSpec((2, _BN, _HALF), lambda i: (0, i, 0)),
            pl.BlockSpec((_BN, 1), lambda i: (i, 0)),
            pl.BlockSpec((1, _D), lambda i: (0, 0)),
        ],
        out_specs=pl.BlockSpec((_BN, _D), lambda i: (i, 0)),
        out_shape=jax.ShapeDtypeStruct((_N, _D), jnp.float32),
    )(acc, dinv, b.reshape(1, _D))

    return out


# speedup vs baseline: 1.2162x; 1.0508x over previous
import functools

import jax
import jax.numpy as jnp
from jax import lax
from jax.experimental import pallas as pl
from jax.experimental.pallas import tpu as pltpu
from jax.experimental.pallas import tpu_sc as plsc

_N = 10000
_D = 256
_E = 160000
_NPAD = 10240
_HALF = 128
_NC = 2
_NS = 16
_K = 128
_NOP = _E // _K
_OPT_E = 80
_OPT_D = 40
_OPROWS = 1280
_ROWS_PER_SUB = _NPAD // _NS
_NSLOT = 2
_OPC = 40
_NCH = _OPT_E // _OPC


def _deg_body(dst2_hbm, out_hbm, didx2, ones, zrow, deg_sh, sem):
    c = lax.axis_index("c")
    s = lax.axis_index("s")
    for u in range(8):
        ones[0, pl.ds(u * 16, 16)] = jnp.full((16,), 1.0, jnp.float32)
        zrow[0, pl.ds(u * 16, 16)] = jnp.zeros((16,), jnp.float32)
    def zbody(m, _):
        pltpu.sync_copy(zrow.at[0], deg_sh.at[pl.ds(s * _ROWS_PER_SUB + m * _HALF, _HALF)])
        return 0
    lax.fori_loop(0, _ROWS_PER_SUB // _HALF, zbody, 0)

    w = s * _NC + c
    base = w * _OPT_D
    nops = jnp.minimum(_OPT_D, _NOP - base)
    pltpu.sync_copy(dst2_hbm.at[pl.ds(base, _OPT_D), :], didx2)
    plsc.subcore_barrier()

    def fire(j, _):
        @pl.when(j < nops)
        def _():
            pltpu.async_copy(ones.at[0], deg_sh.at[didx2.at[j]], sem, add=True)
        return 0
    lax.fori_loop(0, _OPT_D, fire, 0)
    def drain(j, _):
        @pl.when(j < nops)
        def _():
            pltpu.make_async_copy(ones.at[0], deg_sh.at[didx2.at[0]], sem).wait()
        return 0
    lax.fori_loop(0, _OPT_D, drain, 0)
    plsc.subcore_barrier()

    def wbody(m, _):
        off = s * _ROWS_PER_SUB + m * _HALF
        pltpu.sync_copy(deg_sh.at[pl.ds(off, _HALF)], zrow.at[0])
        pltpu.sync_copy(zrow.at[0], out_hbm.at[pl.ds(c * _NPAD + off, _HALF)])
        return 0
    lax.fori_loop(0, _ROWS_PER_SUB // _HALF, wbody, 0)


@functools.partial(
    pl.kernel,
    out_type=jax.ShapeDtypeStruct((_NC * _NPAD,), jnp.float32),
    mesh=plsc.VectorSubcoreMesh(core_axis_name="c", subcore_axis_name="s"),
    scratch_types=[
        pltpu.VMEM((_OPT_D, _K), jnp.int32),
        pltpu.VMEM((1, _HALF), jnp.float32),
        pltpu.VMEM((1, _HALF), jnp.float32),
        pltpu.VMEM_SHARED((_NPAD,), jnp.float32),
        pltpu.SemaphoreType.DMA,
    ],
)
def _deg_call(dst2_hbm, out_hbm, didx2, ones, zrow, deg_sh, sem):
    _deg_body(dst2_hbm, out_hbm, didx2, ones, zrow, deg_sh, sem)


def _ab_tc(x_ref, w_ref, p0_ref, p1_ref, hp2_ref, dinv_ref):
    h = jnp.dot(x_ref[...], w_ref[...], preferred_element_type=jnp.float32)
    deg = p0_ref[...] + p1_ref[...] + 1.0
    dinv = lax.rsqrt(deg)
    hp = h * dinv
    hp2_ref[0] = hp[:, :_HALF]
    hp2_ref[1] = hp[:, _HALF:]
    dinv_ref[...] = dinv


def _edge_body(src2_hbm, dst2_hbm, hcat_hbm, out_hbm, sidx2, didx2, rows, acc_sh, gsems, ssems):
    c = lax.axis_index("c")
    s = lax.axis_index("s")
    base = s * _OPT_E
    nops = jnp.minimum(_OPT_E, _NOP - base)

    def zfill(i, _):
        for u in range(8):
            rows[0][i, pl.ds(u * 16, 16)] = jnp.zeros((16,), jnp.float32)
        return 0
    lax.fori_loop(0, _K, zfill, 0)
    def zbody(m, _):
        pltpu.sync_copy(rows[0], acc_sh.at[pl.ds(s * _ROWS_PER_SUB + m * _K, _K), :])
        return 0
    lax.fori_loop(0, _ROWS_PER_SUB // _K, zbody, 0)

    cbase = c * _N

    def g_start(j, p):
        pltpu.async_copy(hcat_hbm.at[sidx2.at[j]], rows[p], gsems[p])
    def g_wait(p):
        pltpu.make_async_copy(hcat_hbm.at[sidx2.at[0]], rows[p], gsems[p]).wait()
    def s_start(j, p):
        pltpu.async_copy(rows[p], acc_sh.at[didx2.at[j]], ssems[p], add=True)
    def s_wait(p):
        pltpu.make_async_copy(rows[p], acc_sh.at[didx2.at[0]], ssems[p]).wait()

    plsc.subcore_barrier()

    for ch in range(_NCH):
        cb = base + ch * _OPC
        nops = jnp.minimum(_OPC, jnp.maximum(0, _NOP - cb))
        pltpu.sync_copy(src2_hbm.at[pl.ds(cb, _OPC), :], sidx2)
        pltpu.sync_copy(dst2_hbm.at[pl.ds(cb, _OPC), :], didx2)
        def tbody(j, _):
            for u in range(8):
                sidx2[j, pl.ds(u * 16, 16)] = sidx2[j, pl.ds(u * 16, 16)] + cbase
            return 0
        lax.fori_loop(0, _OPC, tbody, 0)

        for q in range(_NSLOT):
            @pl.when(q < nops)
            def _(q=q):
                g_start(q, q)

        def lbody(t, _):
            j0 = t * _NSLOT
            for q in range(_NSLOT):
                @pl.when(j0 + q < nops)
                def _(q=q):
                    g_wait(q)
                    s_start(j0 + q, q)
            for q in range(_NSLOT):
                @pl.when(j0 + q < nops)
                def _(q=q):
                    s_wait(q)
                @pl.when(j0 + q + _NSLOT < nops)
                def _(q=q):
                    g_start(j0 + q + _NSLOT, q)
            return 0
        lax.fori_loop(0, (_OPC + _NSLOT - 1) // _NSLOT, lbody, 0)
    plsc.subcore_barrier()

    def wbody(m, _):
        off = s * _ROWS_PER_SUB + m * _K
        pltpu.sync_copy(acc_sh.at[pl.ds(off, _K), :], rows[0])
        pltpu.sync_copy(rows[0], out_hbm.at[pl.ds(c * _NPAD + off, _K), :])
        return 0
    lax.fori_loop(0, _ROWS_PER_SUB // _K, wbody, 0)


@functools.partial(
    pl.kernel,
    out_type=jax.ShapeDtypeStruct((_NC * _NPAD, _HALF), jnp.float32),
    mesh=plsc.VectorSubcoreMesh(core_axis_name="c", subcore_axis_name="s"),
    scratch_types=[
        pltpu.VMEM((_OPC, _K), jnp.int32),
        pltpu.VMEM((_OPC, _K), jnp.int32),
        [pltpu.VMEM((_K, _HALF), jnp.float32)] * _NSLOT,
        pltpu.VMEM_SHARED((_NPAD, _HALF), jnp.float32),
        [pltpu.SemaphoreType.DMA] * _NSLOT,
        [pltpu.SemaphoreType.DMA] * _NSLOT,
    ],
)
def _edge_call(src2_hbm, dst2_hbm, hcat_hbm, out_hbm, sidx2, didx2, rows, acc_sh, gsems, ssems):
    _edge_body(src2_hbm, dst2_hbm, hcat_hbm, out_hbm, sidx2, didx2, rows, acc_sh, gsems, ssems)


def _fin_tc(acc_ref, hp2_ref, dinv_ref, b_ref, o_ref):
    dinv = dinv_ref[...]
    o0 = (acc_ref[0] + hp2_ref[0]) * dinv
    o1 = (acc_ref[1] + hp2_ref[1]) * dinv
    o_ref[...] = jnp.concatenate([o0, o1], axis=1) + b_ref[...]


_BN = 2000


def kernel(x, edge_index, W, b):
    ei = jnp.pad(edge_index, ((0, 0), (0, _OPROWS * _K - _E)))
    ei = ei.reshape(2, _OPROWS, _K)
    src2 = ei[0]
    dst2 = ei[1]

    degflat = _deg_call(dst2)
    p0 = degflat[:_N].reshape(_N, 1)
    p1 = degflat[_NPAD:_NPAD + _N].reshape(_N, 1)

    hp2, dinv = pl.pallas_call(
        _ab_tc,
        grid=(_N // _BN,),
        in_specs=[
            pl.BlockSpec((_BN, _D), lambda i: (i, 0)),
            pl.BlockSpec((_D, _D), lambda i: (0, 0)),
            pl.BlockSpec((_BN, 1), lambda i: (i, 0)),
            pl.BlockSpec((_BN, 1), lambda i: (i, 0)),
        ],
        out_specs=[
            pl.BlockSpec((2, _BN, _HALF), lambda i: (0, i, 0)),
            pl.BlockSpec((_BN, 1), lambda i: (i, 0)),
        ],
        out_shape=[
            jax.ShapeDtypeStruct((2, _N, _HALF), jnp.float32),
            jax.ShapeDtypeStruct((_N, 1), jnp.float32),
        ],
    )(x, W, p0, p1)

    hcat = hp2.reshape(2 * _N, _HALF)
    accflat = _edge_call(src2, dst2, hcat)
    acc = accflat.reshape(2, _NPAD, _HALF)

    out = pl.pallas_call(
        _fin_tc,
        grid=(_N // _BN,),
        in_specs=[
            pl.BlockSpec((2, _BN, _HALF), lambda i: (0, i, 0)),
            pl.BlockSpec((2, _BN, _HALF), lambda i: (0, i, 0)),
            pl.BlockSpec((_BN, 1), lambda i: (i, 0)),
            pl.BlockSpec((1, _D), lambda i: (0, 0)),
        ],
        out_specs=pl.BlockSpec((_BN, _D), lambda i: (i, 0)),
        out_shape=jax.ShapeDtypeStruct((_N, _D), jnp.float32),
    )(acc, hp2, dinv, b.reshape(1, _D))

    return out


# speedup vs baseline: 1.2257x; 1.0078x over previous
import functools

import jax
import jax.numpy as jnp
from jax import lax
from jax.experimental import pallas as pl
from jax.experimental.pallas import tpu as pltpu
from jax.experimental.pallas import tpu_sc as plsc

_N = 10000
_D = 256
_E = 160000
_NPAD = 10240
_HALF = 128
_NC = 2
_NS = 16
_K = 128
_NOP = _E // _K
_OPT_E = 80
_OPT_D = 40
_OPROWS = 1280
_ROWS_PER_SUB = _NPAD // _NS
_NSLOT = 2
_OPC = 40
_NCH = _OPT_E // _OPC


def _deg_body(dst2_hbm, out_hbm, didx2, ones, zrow, deg_sh, sem):
    c = lax.axis_index("c")
    s = lax.axis_index("s")
    for u in range(8):
        ones[0, pl.ds(u * 16, 16)] = jnp.full((16,), 1.0, jnp.float32)
        zrow[0, pl.ds(u * 16, 16)] = jnp.zeros((16,), jnp.float32)
    def zbody(m, _):
        pltpu.sync_copy(zrow.at[0], deg_sh.at[pl.ds(s * _ROWS_PER_SUB + m * _HALF, _HALF)])
        return 0
    lax.fori_loop(0, _ROWS_PER_SUB // _HALF, zbody, 0)

    w = s * _NC + c
    base = w * _OPT_D
    nops = jnp.minimum(_OPT_D, _NOP - base)
    pltpu.sync_copy(dst2_hbm.at[pl.ds(base, _OPT_D), :], didx2)
    plsc.subcore_barrier()

    def fire(j, _):
        @pl.when(j < nops)
        def _():
            pltpu.async_copy(ones.at[0], deg_sh.at[didx2.at[j]], sem, add=True)
        return 0
    lax.fori_loop(0, _OPT_D, fire, 0)
    def drain(j, _):
        @pl.when(j < nops)
        def _():
            pltpu.make_async_copy(ones.at[0], deg_sh.at[didx2.at[0]], sem).wait()
        return 0
    lax.fori_loop(0, _OPT_D, drain, 0)
    plsc.subcore_barrier()

    def wbody(m, _):
        off = s * _ROWS_PER_SUB + m * _HALF
        pltpu.sync_copy(deg_sh.at[pl.ds(off, _HALF)], zrow.at[0])
        pltpu.sync_copy(zrow.at[0], out_hbm.at[pl.ds(c * _NPAD + off, _HALF)])
        return 0
    lax.fori_loop(0, _ROWS_PER_SUB // _HALF, wbody, 0)


@functools.partial(
    pl.kernel,
    out_type=jax.ShapeDtypeStruct((_NC * _NPAD,), jnp.float32),
    mesh=plsc.VectorSubcoreMesh(core_axis_name="c", subcore_axis_name="s"),
    scratch_types=[
        pltpu.VMEM((_OPT_D, _K), jnp.int32),
        pltpu.VMEM((1, _HALF), jnp.float32),
        pltpu.VMEM((1, _HALF), jnp.float32),
        pltpu.VMEM_SHARED((_NPAD,), jnp.float32),
        pltpu.SemaphoreType.DMA,
    ],
)
def _deg_call(dst2_hbm, out_hbm, didx2, ones, zrow, deg_sh, sem):
    _deg_body(dst2_hbm, out_hbm, didx2, ones, zrow, deg_sh, sem)


def _ab_tc(x_ref, w_ref, p0_ref, p1_ref, hp2_ref, dinv_ref):
    h = jnp.dot(x_ref[...], w_ref[...], preferred_element_type=jnp.float32)
    deg = p0_ref[...] + p1_ref[...] + 1.0
    dinv = lax.rsqrt(deg)
    hp = h * dinv
    hp2_ref[0] = hp[:, :_HALF]
    hp2_ref[1] = hp[:, _HALF:]
    dinv_ref[...] = dinv


def _edge_body(src2_hbm, dst2_hbm, hcat_hbm, out_hbm, sidx2, didx2, rows, acc_sh, gsems, ssems):
    c = lax.axis_index("c")
    s = lax.axis_index("s")
    base = s * _OPT_E
    nops = jnp.minimum(_OPT_E, _NOP - base)

    def zfill(i, _):
        for u in range(8):
            rows[0][i, pl.ds(u * 16, 16)] = jnp.zeros((16,), jnp.float32)
        return 0
    lax.fori_loop(0, _K, zfill, 0)
    def zbody(m, _):
        pltpu.sync_copy(rows[0], acc_sh.at[pl.ds(s * _ROWS_PER_SUB + m * _K, _K), :])
        return 0
    lax.fori_loop(0, _ROWS_PER_SUB // _K, zbody, 0)

    cbase = c * _N

    def g_start(j, p):
        pltpu.async_copy(hcat_hbm.at[sidx2.at[j]], rows[p], gsems[p])
    def g_wait(p):
        pltpu.make_async_copy(hcat_hbm.at[sidx2.at[0]], rows[p], gsems[p]).wait()
    def s_start(j, p):
        pltpu.async_copy(rows[p], acc_sh.at[didx2.at[j]], ssems[p], add=True)
    def s_wait(p):
        pltpu.make_async_copy(rows[p], acc_sh.at[didx2.at[0]], ssems[p]).wait()

    plsc.subcore_barrier()

    for ch in range(_NCH):
        cb = base + ch * _OPC
        nops = jnp.minimum(_OPC, jnp.maximum(0, _NOP - cb))
        pltpu.sync_copy(src2_hbm.at[pl.ds(cb, _OPC), :], sidx2)
        pltpu.sync_copy(dst2_hbm.at[pl.ds(cb, _OPC), :], didx2)
        def tbody(j, _):
            for u in range(8):
                sidx2[j, pl.ds(u * 16, 16)] = sidx2[j, pl.ds(u * 16, 16)] + cbase
            return 0
        lax.fori_loop(0, _OPC, tbody, 0)

        for q in range(_NSLOT):
            @pl.when(q < nops)
            def _(q=q):
                g_start(q, q)

        def lbody(t, _):
            j0 = t * _NSLOT
            for q in range(_NSLOT):
                @pl.when(j0 + q < nops)
                def _(q=q):
                    g_wait(q)
                    s_start(j0 + q, q)
            for q in range(_NSLOT):
                @pl.when(j0 + q < nops)
                def _(q=q):
                    s_wait(q)
                @pl.when(j0 + q + _NSLOT < nops)
                def _(q=q):
                    g_start(j0 + q + _NSLOT, q)
            return 0
        lax.fori_loop(0, (_OPC + _NSLOT - 1) // _NSLOT, lbody, 0)
    plsc.subcore_barrier()

    def wbody(m, _):
        off = s * _ROWS_PER_SUB + m * _K
        pltpu.sync_copy(acc_sh.at[pl.ds(off, _K), :], rows[0])
        pltpu.sync_copy(rows[0], out_hbm.at[pl.ds(c * _NPAD + off, _K), :])
        return 0
    lax.fori_loop(0, _ROWS_PER_SUB // _K, wbody, 0)


@functools.partial(
    pl.kernel,
    out_type=jax.ShapeDtypeStruct((_NC * _NPAD, _HALF), jnp.float32),
    mesh=plsc.VectorSubcoreMesh(core_axis_name="c", subcore_axis_name="s"),
    scratch_types=[
        pltpu.VMEM((_OPC, _K), jnp.int32),
        pltpu.VMEM((_OPC, _K), jnp.int32),
        [pltpu.VMEM((_K, _HALF), jnp.float32)] * _NSLOT,
        pltpu.VMEM_SHARED((_NPAD, _HALF), jnp.float32),
        [pltpu.SemaphoreType.DMA] * _NSLOT,
        [pltpu.SemaphoreType.DMA] * _NSLOT,
    ],
)
def _edge_call(src2_hbm, dst2_hbm, hcat_hbm, out_hbm, sidx2, didx2, rows, acc_sh, gsems, ssems):
    _edge_body(src2_hbm, dst2_hbm, hcat_hbm, out_hbm, sidx2, didx2, rows, acc_sh, gsems, ssems)


def _fin_tc(acc_ref, hp2_ref, dinv_ref, b_ref, o_ref):
    dinv = dinv_ref[...]
    o0 = (acc_ref[0] + hp2_ref[0]) * dinv
    o1 = (acc_ref[1] + hp2_ref[1]) * dinv
    o_ref[...] = jnp.concatenate([o0, o1], axis=1) + b_ref[...]


_BN = 5000


def kernel(x, edge_index, W, b):
    ei = jnp.pad(edge_index, ((0, 0), (0, _OPROWS * _K - _E)))
    ei = ei.reshape(2, _OPROWS, _K)
    src2 = ei[0]
    dst2 = ei[1]

    degflat = _deg_call(dst2)
    p0 = degflat[:_N].reshape(_N, 1)
    p1 = degflat[_NPAD:_NPAD + _N].reshape(_N, 1)

    hp2, dinv = pl.pallas_call(
        _ab_tc,
        grid=(_N // _BN,),
        in_specs=[
            pl.BlockSpec((_BN, _D), lambda i: (i, 0)),
            pl.BlockSpec((_D, _D), lambda i: (0, 0)),
            pl.BlockSpec((_BN, 1), lambda i: (i, 0)),
            pl.BlockSpec((_BN, 1), lambda i: (i, 0)),
        ],
        out_specs=[
            pl.BlockSpec((2, _BN, _HALF), lambda i: (0, i, 0)),
            pl.BlockSpec((_BN, 1), lambda i: (i, 0)),
        ],
        out_shape=[
            jax.ShapeDtypeStruct((2, _N, _HALF), jnp.float32),
            jax.ShapeDtypeStruct((_N, 1), jnp.float32),
        ],
    )(x, W, p0, p1)

    hcat = hp2.reshape(2 * _N, _HALF)
    accflat = _edge_call(src2, dst2, hcat)
    acc = accflat.reshape(2, _NPAD, _HALF)

    out = pl.pallas_call(
        _fin_tc,
        grid=(_N // _BN,),
        in_specs=[
            pl.BlockSpec((2, _BN, _HALF), lambda i: (0, i, 0)),
            pl.BlockSpec((2, _BN, _HALF), lambda i: (0, i, 0)),
            pl.BlockSpec((_BN, 1), lambda i: (i, 0)),
            pl.BlockSpec((1, _D), lambda i: (0, 0)),
        ],
        out_specs=pl.BlockSpec((_BN, _D), lambda i: (i, 0)),
        out_shape=jax.ShapeDtypeStruct((_N, _D), jnp.float32),
    )(acc, hp2, dinv, b.reshape(1, _D))

    return out
